# baseline (speedup 1.0000x reference)
import jax
import jax.numpy as jnp
from jax.experimental import pallas as pl
from jax.experimental.pallas import tpu as pltpu

_ROWS = 296


def _csd_block(x_ref, y_ref, ld_ref, lf_ref, w_ref, wbg_ref, wt_ref,
               e4_ref, out_ref):
    x = x_ref[...]
    y = y_ref[...]
    ld = ld_ref[...]
    lf = lf_ref[...]
    w = w_ref[...]
    wbg = wbg_ref[...]
    wt = wt_ref[...]
    e4 = e4_ref[...]

    bg = jnp.dot(x, wbg)
    bgx = jnp.dot(bg, wt)
    cmpf = jnp.where(x > bgx, 1.0, 0.0)
    cnt = jnp.dot(cmpf, w)
    maskf = jnp.where(cnt > 0.0, 1.0, 0.0)

    p = x + 1e-7
    q = y + 1e-7
    t = (p - q) * jnp.log(p / q)
    rkl = jnp.dot(t, w)
    conf_p = jnp.sum(maskf * rkl)

    lane = jax.lax.broadcasted_iota(jnp.int32, (1, 128), 1)
    ind0 = jnp.where(lane % 4 == 0, 1.0, 0.0)
    d = ld - lf
    dd = d * d + 4.0 * ld * lf * ind0
    mexp = jnp.dot(maskf, e4)
    loc_p = jnp.sum(mexp * dd)

    cnt_p = jnp.sum(maskf)

    partial = jnp.stack([cnt_p, conf_p, loc_p]).reshape(1, 3)

    @pl.when(pl.program_id(0) == 0)
    def _():
        out_ref[...] = partial

    @pl.when(pl.program_id(0) != 0)
    def _():
        out_ref[...] = out_ref[...] + partial


def kernel(conf, conf_flip, loc, loc_flip):
    b, num_p, c = conf.shape
    groups = b * num_p
    gpr = 32
    width = c * gpr
    rows = groups // gpr
    n_blocks = rows // _ROWS

    x2 = conf.reshape(rows, width)
    y2 = conf_flip.reshape(rows, width)
    l2 = loc.reshape(rows, 4 * gpr)
    f2 = loc_flip.reshape(rows, 4 * gpr)

    lane_c = jnp.arange(width, dtype=jnp.int32)
    grp = jnp.arange(gpr, dtype=jnp.int32)
    w = (lane_c[:, None] // c == grp[None, :]).astype(jnp.float32)
    wbg = (lane_c[:, None] == c * grp[None, :]).astype(jnp.float32)
    wt = w.T
    lane4 = jnp.arange(4 * gpr, dtype=jnp.int32)
    e4 = (grp[:, None] == lane4[None, :] // 4).astype(jnp.float32)

    out = pl.pallas_call(
        _csd_block,
        grid=(n_blocks,),
        in_specs=[
            pl.BlockSpec((_ROWS, width), lambda i: (i, 0)),
            pl.BlockSpec((_ROWS, width), lambda i: (i, 0)),
            pl.BlockSpec((_ROWS, 4 * gpr), lambda i: (i, 0)),
            pl.BlockSpec((_ROWS, 4 * gpr), lambda i: (i, 0)),
            pl.BlockSpec((width, gpr), lambda i: (0, 0)),
            pl.BlockSpec((width, gpr), lambda i: (0, 0)),
            pl.BlockSpec((gpr, width), lambda i: (0, 0)),
            pl.BlockSpec((gpr, 4 * gpr), lambda i: (0, 0)),
        ],
        out_specs=pl.BlockSpec((1, 3), lambda i: (0, 0)),
        out_shape=jax.ShapeDtypeStruct((1, 3), jnp.float32),
        compiler_params=pltpu.CompilerParams(
            dimension_semantics=("arbitrary",),
        ),
    )(x2, y2, l2, f2, w, wbg, wt, e4)

    total = jnp.maximum(out[0, 0], 1.0)
    return out[0, 1] / (2.0 * total) + out[0, 2] / (4.0 * total)


# speedup vs baseline: 58.1217x; 58.1217x over previous
import jax
import jax.numpy as jnp
from jax.experimental import pallas as pl
from jax.experimental.pallas import tpu as pltpu

_BB = 8
_PB = 2944


def _make_body(num_p):
    def _body(x_ref, y_ref, l_ref, f_ref, out_ref, acc_m, acc_c, acc_l):
        bi = pl.program_id(0)
        pj = pl.program_id(1)
        nbi = pl.num_programs(0)
        npj = pl.num_programs(1)

        x = x_ref[...]
        y = y_ref[...]
        l = l_ref[...]
        f = f_ref[...]

        lane = jax.lax.broadcasted_iota(jnp.int32, (_BB, _PB), 1)
        valid = (pj * _PB + lane) < num_p

        bg = x[0]
        fg = jnp.max(x[1:], axis=0)
        mb = (fg > bg) & valid

        p = x + 1e-7
        q = y + 1e-7
        tsum = jnp.sum((p - q) * (jnp.log(p) - jnp.log(q)), axis=0)

        d = l - f
        rloc = jnp.sum(d * d, axis=1) + 4.0 * l[:, 0] * f[:, 0]

        m_c = jnp.where(mb, 1.0, 0.0)
        c_c = jnp.where(mb, tsum, 0.0)
        l_c = jnp.where(mb, rloc, 0.0)

        first = (bi == 0) & (pj == 0)

        @pl.when(first)
        def _():
            acc_m[...] = m_c
            acc_c[...] = c_c
            acc_l[...] = l_c

        @pl.when(jnp.logical_not(first))
        def _():
            acc_m[...] = acc_m[...] + m_c
            acc_c[...] = acc_c[...] + c_c
            acc_l[...] = acc_l[...] + l_c

        @pl.when((bi == nbi - 1) & (pj == npj - 1))
        def _():
            out_ref[...] = jnp.stack([
                jnp.sum(acc_m[...]),
                jnp.sum(acc_c[...]),
                jnp.sum(acc_l[...]),
            ]).reshape(1, 3)

    return _body


def kernel(conf, conf_flip, loc, loc_flip):
    b, num_p, c = conf.shape

    xt = conf.transpose(2, 0, 1)
    yt = conf_flip.transpose(2, 0, 1)
    lt = loc.transpose(0, 2, 1)
    ft = loc_flip.transpose(0, 2, 1)

    grid = (b // _BB, -(-num_p // _PB))
    out = pl.pallas_call(
        _make_body(num_p),
        grid=grid,
        in_specs=[
            pl.BlockSpec((c, _BB, _PB), lambda i, j: (0, i, j)),
            pl.BlockSpec((c, _BB, _PB), lambda i, j: (0, i, j)),
            pl.BlockSpec((_BB, 4, _PB), lambda i, j: (i, 0, j)),
            pl.BlockSpec((_BB, 4, _PB), lambda i, j: (i, 0, j)),
        ],
        out_specs=pl.BlockSpec((1, 3), lambda i, j: (0, 0)),
        out_shape=jax.ShapeDtypeStruct((1, 3), jnp.float32),
        scratch_shapes=[
            pltpu.VMEM((_BB, _PB), jnp.float32),
            pltpu.VMEM((_BB, _PB), jnp.float32),
            pltpu.VMEM((_BB, _PB), jnp.float32),
        ],
        compiler_params=pltpu.CompilerParams(
            dimension_semantics=("arbitrary", "arbitrary"),
        ),
    )(xt, yt, lt, ft)

    total = jnp.maximum(out[0, 0], 1.0)
    return out[0, 1] / (2.0 * total) + out[0, 2] / (4.0 * total)


# speedup vs baseline: 59.3737x; 1.0215x over previous
import jax
import jax.numpy as jnp
from jax.experimental import pallas as pl
from jax.experimental.pallas import tpu as pltpu

_BB = 16
_PB = 2944


def _make_body(num_p):
    def _body(x_ref, y_ref, l_ref, f_ref, out_ref, acc_m, acc_c, acc_l):
        bi = pl.program_id(0)
        pj = pl.program_id(1)
        nbi = pl.num_programs(0)
        npj = pl.num_programs(1)

        x = x_ref[...]
        y = y_ref[...]
        l = l_ref[...]
        f = f_ref[...]

        lane = jax.lax.broadcasted_iota(jnp.int32, (_BB, _PB), 1)
        valid = (pj * _PB + lane) < num_p

        bg = x[0]
        fg = jnp.max(x[1:], axis=0)
        mb = (fg > bg) & valid

        p = x + 1e-7
        q = y + 1e-7
        tsum = jnp.sum((p - q) * (jnp.log(p) - jnp.log(q)), axis=0)

        d = l - f
        rloc = jnp.sum(d * d, axis=1) + 4.0 * l[:, 0] * f[:, 0]

        m_c = jnp.where(mb, 1.0, 0.0)
        c_c = jnp.where(mb, tsum, 0.0)
        l_c = jnp.where(mb, rloc, 0.0)

        first = (bi == 0) & (pj == 0)

        @pl.when(first)
        def _():
            acc_m[...] = m_c
            acc_c[...] = c_c
            acc_l[...] = l_c

        @pl.when(jnp.logical_not(first))
        def _():
            acc_m[...] = acc_m[...] + m_c
            acc_c[...] = acc_c[...] + c_c
            acc_l[...] = acc_l[...] + l_c

        @pl.when((bi == nbi - 1) & (pj == npj - 1))
        def _():
            out_ref[...] = jnp.stack([
                jnp.sum(acc_m[...]),
                jnp.sum(acc_c[...]),
                jnp.sum(acc_l[...]),
            ]).reshape(1, 3)

    return _body


def kernel(conf, conf_flip, loc, loc_flip):
    b, num_p, c = conf.shape

    xt = conf.transpose(2, 0, 1)
    yt = conf_flip.transpose(2, 0, 1)
    lt = loc.transpose(0, 2, 1)
    ft = loc_flip.transpose(0, 2, 1)

    grid = (b // _BB, -(-num_p // _PB))
    out = pl.pallas_call(
        _make_body(num_p),
        grid=grid,
        in_specs=[
            pl.BlockSpec((c, _BB, _PB), lambda i, j: (0, i, j)),
            pl.BlockSpec((c, _BB, _PB), lambda i, j: (0, i, j)),
            pl.BlockSpec((_BB, 4, _PB), lambda i, j: (i, 0, j)),
            pl.BlockSpec((_BB, 4, _PB), lambda i, j: (i, 0, j)),
        ],
        out_specs=pl.BlockSpec((1, 3), lambda i, j: (0, 0)),
        out_shape=jax.ShapeDtypeStruct((1, 3), jnp.float32),
        scratch_shapes=[
            pltpu.VMEM((_BB, _PB), jnp.float32),
            pltpu.VMEM((_BB, _PB), jnp.float32),
            pltpu.VMEM((_BB, _PB), jnp.float32),
        ],
        compiler_params=pltpu.CompilerParams(
            dimension_semantics=("arbitrary", "arbitrary"),
        ),
    )(xt, yt, lt, ft)

    total = jnp.maximum(out[0, 0], 1.0)
    return out[0, 1] / (2.0 * total) + out[0, 2] / (4.0 * total)
